# baseline (speedup 1.0000x reference)
import jax
import jax.numpy as jnp
from jax import lax
from jax.experimental import pallas as pl
from jax.experimental.pallas import tpu as pltpu
from jax.experimental.pallas import tpu_sc as plsc

_B, _N, _G, _K = 64, 2048, 128, 32
_L = 16
_NC, _NS = 2, 16
_NW = _NC * _NS
_BPW = _B // _NW
_NCH = _N // _L
_GCH = _G // _L
_BIG = 1e10
_HUGE = 3.0e38


def _bf16r(v):
    u = lax.bitcast_convert_type(v, jnp.uint32)
    u = (u + jnp.uint32(0x7FFF) + ((u >> jnp.uint32(16)) & jnp.uint32(1))) \
        & jnp.uint32(0xFFFF0000)
    return lax.bitcast_convert_type(u, jnp.float32)


def _sc_body(x_hbm, y_hbm, z_hbm, nbr_hbm, cen_hbm,
             x_v, y_v, z_v, dist_v, pn_v,
             cx_v, cy_v, cz_v, cn_v, vis_v, ord_v,
             bx_v, by_v, bz_v, bcx_v, bcy_v, bcz_v,
             nout_v, cout_v):
    wid = lax.axis_index("s") * _NC + lax.axis_index("c")
    iota = lax.iota(jnp.int32, _L)
    lane0 = iota == 0

    def splat_i(i):
        return jnp.full((_L,), i, jnp.int32)

    def gat(ref, idx_vec):
        return plsc.load_gather(ref, [idx_vec])

    def put1(ref, i, val_vec):
        plsc.store_scatter(ref, [splat_i(i)], val_vec, mask=lane0)

    def batch_body(kb, carry0):
        b = wid * _BPW + kb
        pltpu.sync_copy(x_hbm.at[b], x_v)
        pltpu.sync_copy(y_hbm.at[b], y_v)
        pltpu.sync_copy(z_hbm.at[b], z_v)

        def init_chunk(c, carry):
            dist_v[pl.ds(c * _L, _L)] = jnp.full((_L,), _BIG, jnp.float32)
            return carry
        lax.fori_loop(0, _NCH, init_chunk, 0, unroll=8)

        def fps_step(i, far):
            fari = splat_i(far)
            cx = gat(x_v, fari)
            cy = gat(y_v, fari)
            cz = gat(z_v, fari)
            put1(cx_v, i, cx)
            put1(cy_v, i, cy)
            put1(cz_v, i, cz)

            _NACC = 4

            def chunk(c, carry):
                out = []
                for q in range(_NACC):
                    best, bidx = carry[q]
                    o = (c * _NACC + q) * _L
                    dx = x_v[pl.ds(o, _L)] - cx
                    dy = y_v[pl.ds(o, _L)] - cy
                    dz = z_v[pl.ds(o, _L)] - cz
                    d = (dx * dx + dy * dy) + dz * dz
                    nd = jnp.minimum(dist_v[pl.ds(o, _L)], d)
                    dist_v[pl.ds(o, _L)] = nd
                    m = nd > best
                    best = jnp.where(m, nd, best)
                    bidx = jnp.where(m, iota + o, bidx)
                    out.append((best, bidx))
                return tuple(out)

            acc0 = (jnp.full((_L,), -1.0, jnp.float32),
                    jnp.zeros((_L,), jnp.int32))
            accs = lax.fori_loop(0, _NCH // _NACC, chunk, (acc0,) * _NACC,
                                 unroll=2)
            bb = jnp.maximum(jnp.maximum(accs[0][0], accs[1][0]),
                             jnp.maximum(accs[2][0], accs[3][0]))
            mx = jnp.max(bb)
            cand = jnp.minimum(
                jnp.minimum(
                    jnp.where(accs[0][0] == mx, accs[0][1], jnp.int32(_N)),
                    jnp.where(accs[1][0] == mx, accs[1][1], jnp.int32(_N))),
                jnp.minimum(
                    jnp.where(accs[2][0] == mx, accs[2][1], jnp.int32(_N)),
                    jnp.where(accs[3][0] == mx, accs[3][1], jnp.int32(_N))))
            return jnp.min(cand)

        lax.fori_loop(0, _G, fps_step, jnp.int32(0))

        def cn_chunk(c, carry):
            o = c * _L
            a = cx_v[pl.ds(o, _L)]
            bb = cy_v[pl.ds(o, _L)]
            cc = cz_v[pl.ds(o, _L)]
            cn_v[pl.ds(o, _L)] = (a * a + bb * bb) + cc * cc
            return carry
        lax.fori_loop(0, _GCH, cn_chunk, 0, unroll=4)

        def pn_chunk(c, carry):
            o = c * _L
            a = x_v[pl.ds(o, _L)]
            bb = y_v[pl.ds(o, _L)]
            cc = z_v[pl.ds(o, _L)]
            pn_v[pl.ds(o, _L)] = (a * a + bb * bb) + cc * cc
            bx_v[pl.ds(o, _L)] = _bf16r(a)
            by_v[pl.ds(o, _L)] = _bf16r(bb)
            bz_v[pl.ds(o, _L)] = _bf16r(cc)
            return carry
        lax.fori_loop(0, _NCH, pn_chunk, 0, unroll=8)

        def bc_chunk(c, carry):
            o = c * _L
            bcx_v[pl.ds(o, _L)] = _bf16r(cx_v[pl.ds(o, _L)])
            bcy_v[pl.ds(o, _L)] = _bf16r(cy_v[pl.ds(o, _L)])
            bcz_v[pl.ds(o, _L)] = _bf16r(cz_v[pl.ds(o, _L)])
            return carry
        lax.fori_loop(0, _GCH, bc_chunk, 0, unroll=4)

        def vis_chunk(c, carry):
            vis_v[pl.ds(c * _L, _L)] = jnp.zeros((_L,), jnp.float32)
            return carry
        lax.fori_loop(0, _GCH, vis_chunk, 0, unroll=4)
        put1(vis_v, 0, jnp.full((_L,), _HUGE, jnp.float32))
        put1(ord_v, 0, jnp.zeros((_L,), jnp.int32))

        def morton_step(i, cur):
            curi = splat_i(cur)
            ccx = gat(bcx_v, curi)
            ccy = gat(bcy_v, curi)
            ccz = gat(bcz_v, curi)
            ccn = gat(cn_v, curi)

            def chunk(c, carry):
                best, bidx = carry
                o = c * _L
                gx = bcx_v[pl.ds(o, _L)]
                gy = bcy_v[pl.ds(o, _L)]
                gz = bcz_v[pl.ds(o, _L)]
                dot = (ccx * gx + ccy * gy) + ccz * gz
                d2 = (ccn + cn_v[pl.ds(o, _L)]) - 2.0 * dot
                d2 = jnp.maximum(d2, 0.0)
                val = jnp.where(vis_v[pl.ds(o, _L)] > 0.0, _HUGE, d2)
                m = val < best
                best = jnp.where(m, val, best)
                bidx = jnp.where(m, iota + o, bidx)
                return best, bidx

            best, bidx = lax.fori_loop(
                0, _GCH, chunk,
                (jnp.full((_L,), jnp.inf, jnp.float32),
                 jnp.zeros((_L,), jnp.int32)), unroll=4)
            mn = jnp.min(best)
            cand = jnp.where(best == mn, bidx, jnp.int32(_G))
            nxt = jnp.min(cand)
            put1(vis_v, nxt, jnp.full((_L,), _HUGE, jnp.float32))
            put1(ord_v, i, splat_i(nxt))
            return nxt

        lax.fori_loop(1, _G, morton_step, jnp.int32(0))

        def knn_one(gp, carry0):
            gvec = gat(ord_v, splat_i(gp))
            gx = gat(cx_v, gvec)
            gy = gat(cy_v, gvec)
            gz = gat(cz_v, gvec)
            gn = gat(cn_v, gvec)
            gxr = gat(bcx_v, gvec)
            gyr = gat(bcy_v, gvec)
            gzr = gat(bcz_v, gvec)

            def merge16(buf, d2, idx):
                s0, s1, i0, i1 = buf
                dk, ik = plsc.sort_key_val(d2, idx)
                rdk = lax.rev(dk, (0,))
                rik = lax.rev(ik, (0,))
                mm = s1 < rdk
                lk = jnp.where(mm, s1, rdk)
                li = jnp.where(mm, i1, rik)
                lk, li = plsc.sort_key_val(lk, li)
                rlk = lax.rev(lk, (0,))
                rli = lax.rev(li, (0,))
                mm2 = s0 < rlk
                lo_k = jnp.where(mm2, s0, rlk)
                lo_i = jnp.where(mm2, i0, rli)
                hi_k = jnp.where(mm2, rlk, s0)
                hi_i = jnp.where(mm2, rli, i0)
                s0n, i0n = plsc.sort_key_val(lo_k, lo_i)
                s1n, i1n = plsc.sort_key_val(hi_k, hi_i)
                return (s0n, s1n, i0n, i1n)

            _NBUF = 4

            def group(c4, bufs):
                out = []
                for q in range(_NBUF):
                    o = (c4 * _NBUF + q) * _L
                    xv = bx_v[pl.ds(o, _L)]
                    yv = by_v[pl.ds(o, _L)]
                    zv = bz_v[pl.ds(o, _L)]
                    dot = (gxr * xv + gyr * yv) + gzr * zv
                    d2 = (gn + pn_v[pl.ds(o, _L)]) - 2.0 * dot
                    d2 = jnp.maximum(d2, 0.0)
                    out.append(merge16(bufs[q], d2, iota + o))
                return tuple(out)

            buf0 = (jnp.full((_L,), _HUGE, jnp.float32),
                    jnp.full((_L,), _HUGE, jnp.float32),
                    jnp.zeros((_L,), jnp.int32),
                    jnp.zeros((_L,), jnp.int32))
            bufs = lax.fori_loop(0, _NCH // _NBUF, group, (buf0,) * _NBUF)

            def pairmerge(a, b):
                a0, a1, ai0, ai1 = a
                b0, b1, bi0, bi1 = b
                rb0 = lax.rev(b0, (0,))
                rb1 = lax.rev(b1, (0,))
                rbi0 = lax.rev(bi0, (0,))
                rbi1 = lax.rev(bi1, (0,))
                mm0 = a0 < rb1
                m0 = jnp.where(mm0, a0, rb1)
                mi0 = jnp.where(mm0, ai0, rbi1)
                mm1 = a1 < rb0
                m1 = jnp.where(mm1, a1, rb0)
                mi1 = jnp.where(mm1, ai1, rbi0)
                cc = m0 < m1
                lo = jnp.where(cc, m0, m1)
                loi = jnp.where(cc, mi0, mi1)
                hi = jnp.where(cc, m1, m0)
                hii = jnp.where(cc, mi1, mi0)
                s0n, i0n = plsc.sort_key_val(lo, loi)
                s1n, i1n = plsc.sort_key_val(hi, hii)
                return (s0n, s1n, i0n, i1n)

            mA = pairmerge(bufs[0], bufs[1])
            mB = pairmerge(bufs[2], bufs[3])
            s0, s1, i0, i1 = pairmerge(mA, mB)

            base = gp * (_K * 3)
            pos0 = base + 3 * iota
            nx0 = plsc.load_gather(x_v, [i0]) - gx
            ny0 = plsc.load_gather(y_v, [i0]) - gy
            nz0 = plsc.load_gather(z_v, [i0]) - gz
            plsc.store_scatter(nout_v, [pos0], nx0)
            plsc.store_scatter(nout_v, [pos0 + 1], ny0)
            plsc.store_scatter(nout_v, [pos0 + 2], nz0)
            pos1 = base + 3 * _L + 3 * iota
            nx1 = plsc.load_gather(x_v, [i1]) - gx
            ny1 = plsc.load_gather(y_v, [i1]) - gy
            nz1 = plsc.load_gather(z_v, [i1]) - gz
            plsc.store_scatter(nout_v, [pos1], nx1)
            plsc.store_scatter(nout_v, [pos1 + 1], ny1)
            plsc.store_scatter(nout_v, [pos1 + 2], nz1)
            cval = jnp.where(iota == 0, gx, jnp.where(iota == 1, gy, gz))
            plsc.store_scatter(cout_v, [3 * gp + iota], cval,
                               mask=iota < jnp.int32(3))
            return carry0

        lax.fori_loop(0, _G, knn_one, 0)

        pltpu.sync_copy(nout_v, nbr_hbm.at[b])
        pltpu.sync_copy(cout_v, cen_hbm.at[b])
        return carry0

    lax.fori_loop(0, _BPW, batch_body, 0)


import functools


@functools.cache
def _build_sc_call():
    mesh = plsc.VectorSubcoreMesh(core_axis_name="c", subcore_axis_name="s",
                                  num_cores=_NC, num_subcores=_NS)
    return pl.kernel(
        _sc_body,
        compiler_params=pltpu.CompilerParams(needs_layout_passes=False),
        out_type=(jax.ShapeDtypeStruct((_B, _G * _K * 3), jnp.float32),
                  jax.ShapeDtypeStruct((_B, _G * 3), jnp.float32)),
        mesh=mesh,
        scratch_types=[
            pltpu.VMEM((_N,), jnp.float32),
            pltpu.VMEM((_N,), jnp.float32),
            pltpu.VMEM((_N,), jnp.float32),
            pltpu.VMEM((_N,), jnp.float32),
            pltpu.VMEM((_N,), jnp.float32),
            pltpu.VMEM((_G,), jnp.float32),
            pltpu.VMEM((_G,), jnp.float32),
            pltpu.VMEM((_G,), jnp.float32),
            pltpu.VMEM((_G,), jnp.float32),
            pltpu.VMEM((_G,), jnp.float32),
            pltpu.VMEM((_G,), jnp.int32),
            pltpu.VMEM((_N,), jnp.float32),
            pltpu.VMEM((_N,), jnp.float32),
            pltpu.VMEM((_N,), jnp.float32),
            pltpu.VMEM((_G,), jnp.float32),
            pltpu.VMEM((_G,), jnp.float32),
            pltpu.VMEM((_G,), jnp.float32),
            pltpu.VMEM((_G * _K * 3,), jnp.float32),
            pltpu.VMEM((_G * 3,), jnp.float32),
        ],
    )


def kernel(xyz):
    x = xyz[:, :, 0]
    y = xyz[:, :, 1]
    z = xyz[:, :, 2]
    nbr, cen = _build_sc_call()(x, y, z)
    return (nbr.reshape(_B, _G, _K, 3), cen.reshape(_B, _G, 3))


# speedup vs baseline: 1.6357x; 1.6357x over previous
import jax
import jax.numpy as jnp
from jax import lax
from jax.experimental import pallas as pl
from jax.experimental.pallas import tpu as pltpu
from jax.experimental.pallas import tpu_sc as plsc

_B, _N, _G, _K = 64, 2048, 128, 32
_L = 16
_NC, _NS = 2, 16
_NW = _NC * _NS
_BPW = _B // _NW
_NCH = _N // _L
_GCH = _G // _L
_BIG = 1e10
_HUGE = 3.0e38


def _bf16r(v):
    u = lax.bitcast_convert_type(v, jnp.uint32)
    u = (u + jnp.uint32(0x7FFF) + ((u >> jnp.uint32(16)) & jnp.uint32(1))) \
        & jnp.uint32(0xFFFF0000)
    return lax.bitcast_convert_type(u, jnp.float32)


def _sc_body(x_hbm, y_hbm, z_hbm, nbr_hbm, cen_hbm,
             x_v, y_v, z_v, dist_v, pn_v,
             cx_v, cy_v, cz_v, cn_v, vis_v, ord_v,
             bx_v, by_v, bz_v, bcx_v, bcy_v, bcz_v,
             nout_v, cout_v):
    wid = lax.axis_index("s") * _NC + lax.axis_index("c")
    iota = lax.iota(jnp.int32, _L)
    lane0 = iota == 0

    def splat_i(i):
        return jnp.full((_L,), i, jnp.int32)

    def gat(ref, idx_vec):
        return plsc.load_gather(ref, [idx_vec])

    def put1(ref, i, val_vec):
        plsc.store_scatter(ref, [splat_i(i)], val_vec, mask=lane0)

    def batch_body(kb, carry0):
        b = wid * _BPW + kb
        pltpu.sync_copy(x_hbm.at[b], x_v)
        pltpu.sync_copy(y_hbm.at[b], y_v)
        pltpu.sync_copy(z_hbm.at[b], z_v)

        @plsc.parallel_loop(0, _NCH, unroll=8)
        def _init_chunk(c):
            dist_v[pl.ds(c * _L, _L)] = jnp.full((_L,), _BIG, jnp.float32)

        def fps_step(i, far):
            fari = splat_i(far)
            cx = gat(x_v, fari)
            cy = gat(y_v, fari)
            cz = gat(z_v, fari)
            put1(cx_v, i, cx)
            put1(cy_v, i, cy)
            put1(cz_v, i, cz)

            _NACC = 4

            def chunk(c, carry):
                out = []
                for q in range(_NACC):
                    best, bidx = carry[q]
                    o = (c * _NACC + q) * _L
                    dx = x_v[pl.ds(o, _L)] - cx
                    dy = y_v[pl.ds(o, _L)] - cy
                    dz = z_v[pl.ds(o, _L)] - cz
                    d = (dx * dx + dy * dy) + dz * dz
                    nd = jnp.minimum(dist_v[pl.ds(o, _L)], d)
                    dist_v[pl.ds(o, _L)] = nd
                    m = nd > best
                    best = jnp.where(m, nd, best)
                    bidx = jnp.where(m, iota + o, bidx)
                    out.append((best, bidx))
                return tuple(out)

            acc0 = (jnp.full((_L,), -1.0, jnp.float32),
                    jnp.zeros((_L,), jnp.int32))
            accs = plsc.parallel_loop(0, _NCH // _NACC, unroll=2,
                                      carry=(acc0,) * _NACC)(chunk)
            bb = jnp.maximum(jnp.maximum(accs[0][0], accs[1][0]),
                             jnp.maximum(accs[2][0], accs[3][0]))
            mx = jnp.max(bb)
            cand = jnp.minimum(
                jnp.minimum(
                    jnp.where(accs[0][0] == mx, accs[0][1], jnp.int32(_N)),
                    jnp.where(accs[1][0] == mx, accs[1][1], jnp.int32(_N))),
                jnp.minimum(
                    jnp.where(accs[2][0] == mx, accs[2][1], jnp.int32(_N)),
                    jnp.where(accs[3][0] == mx, accs[3][1], jnp.int32(_N))))
            return jnp.min(cand)

        lax.fori_loop(0, _G, fps_step, jnp.int32(0))

        def cn_chunk(c, carry):
            o = c * _L
            a = cx_v[pl.ds(o, _L)]
            bb = cy_v[pl.ds(o, _L)]
            cc = cz_v[pl.ds(o, _L)]
            cn_v[pl.ds(o, _L)] = (a * a + bb * bb) + cc * cc
            return carry
        plsc.parallel_loop(0, _GCH, unroll=4)(
            lambda c: cn_chunk(c, 0) and None)

        def pn_chunk(c, carry):
            o = c * _L
            a = x_v[pl.ds(o, _L)]
            bb = y_v[pl.ds(o, _L)]
            cc = z_v[pl.ds(o, _L)]
            pn_v[pl.ds(o, _L)] = (a * a + bb * bb) + cc * cc
            bx_v[pl.ds(o, _L)] = _bf16r(a)
            by_v[pl.ds(o, _L)] = _bf16r(bb)
            bz_v[pl.ds(o, _L)] = _bf16r(cc)
            return carry
        plsc.parallel_loop(0, _NCH, unroll=8)(
            lambda c: pn_chunk(c, 0) and None)

        def bc_chunk(c, carry):
            o = c * _L
            bcx_v[pl.ds(o, _L)] = _bf16r(cx_v[pl.ds(o, _L)])
            bcy_v[pl.ds(o, _L)] = _bf16r(cy_v[pl.ds(o, _L)])
            bcz_v[pl.ds(o, _L)] = _bf16r(cz_v[pl.ds(o, _L)])
            return carry
        plsc.parallel_loop(0, _GCH, unroll=4)(
            lambda c: bc_chunk(c, 0) and None)

        def vis_chunk(c, carry):
            vis_v[pl.ds(c * _L, _L)] = jnp.zeros((_L,), jnp.float32)
            return carry
        plsc.parallel_loop(0, _GCH, unroll=4)(
            lambda c: vis_chunk(c, 0) and None)
        put1(vis_v, 0, jnp.full((_L,), _HUGE, jnp.float32))
        put1(ord_v, 0, jnp.zeros((_L,), jnp.int32))

        def morton_step(i, cur):
            curi = splat_i(cur)
            ccx = gat(bcx_v, curi)
            ccy = gat(bcy_v, curi)
            ccz = gat(bcz_v, curi)
            ccn = gat(cn_v, curi)

            def chunk(c, carry):
                best, bidx = carry
                o = c * _L
                gx = bcx_v[pl.ds(o, _L)]
                gy = bcy_v[pl.ds(o, _L)]
                gz = bcz_v[pl.ds(o, _L)]
                dot = (ccx * gx + ccy * gy) + ccz * gz
                d2 = (ccn + cn_v[pl.ds(o, _L)]) - 2.0 * dot
                d2 = jnp.maximum(d2, 0.0)
                val = jnp.where(vis_v[pl.ds(o, _L)] > 0.0, _HUGE, d2)
                m = val < best
                best = jnp.where(m, val, best)
                bidx = jnp.where(m, iota + o, bidx)
                return best, bidx

            best, bidx = plsc.parallel_loop(
                0, _GCH, unroll=4,
                carry=(jnp.full((_L,), jnp.inf, jnp.float32),
                       jnp.zeros((_L,), jnp.int32)))(chunk)
            mn = jnp.min(best)
            cand = jnp.where(best == mn, bidx, jnp.int32(_G))
            nxt = jnp.min(cand)
            put1(vis_v, nxt, jnp.full((_L,), _HUGE, jnp.float32))
            put1(ord_v, i, splat_i(nxt))
            return nxt

        lax.fori_loop(1, _G, morton_step, jnp.int32(0))

        def knn_one(gp, carry0):
            gvec = gat(ord_v, splat_i(gp))
            gx = gat(cx_v, gvec)
            gy = gat(cy_v, gvec)
            gz = gat(cz_v, gvec)
            gn = gat(cn_v, gvec)
            gxr = gat(bcx_v, gvec)
            gyr = gat(bcy_v, gvec)
            gzr = gat(bcz_v, gvec)

            def merge16(buf, d2, idx):
                s0, s1, i0, i1 = buf
                dk, ik = plsc.sort_key_val(d2, idx)
                rdk = lax.rev(dk, (0,))
                rik = lax.rev(ik, (0,))
                mm = s1 < rdk
                lk = jnp.where(mm, s1, rdk)
                li = jnp.where(mm, i1, rik)
                lk, li = plsc.sort_key_val(lk, li)
                rlk = lax.rev(lk, (0,))
                rli = lax.rev(li, (0,))
                mm2 = s0 < rlk
                lo_k = jnp.where(mm2, s0, rlk)
                lo_i = jnp.where(mm2, i0, rli)
                hi_k = jnp.where(mm2, rlk, s0)
                hi_i = jnp.where(mm2, rli, i0)
                s0n, i0n = plsc.sort_key_val(lo_k, lo_i)
                s1n, i1n = plsc.sort_key_val(hi_k, hi_i)
                return (s0n, s1n, i0n, i1n)

            _NBUF = 4

            def group(c4, bufs):
                out = []
                for q in range(_NBUF):
                    o = (c4 * _NBUF + q) * _L
                    xv = bx_v[pl.ds(o, _L)]
                    yv = by_v[pl.ds(o, _L)]
                    zv = bz_v[pl.ds(o, _L)]
                    dot = (gxr * xv + gyr * yv) + gzr * zv
                    d2 = (gn + pn_v[pl.ds(o, _L)]) - 2.0 * dot
                    d2 = jnp.maximum(d2, 0.0)
                    out.append(merge16(bufs[q], d2, iota + o))
                return tuple(out)

            buf0 = (jnp.full((_L,), _HUGE, jnp.float32),
                    jnp.full((_L,), _HUGE, jnp.float32),
                    jnp.zeros((_L,), jnp.int32),
                    jnp.zeros((_L,), jnp.int32))
            bufs = plsc.parallel_loop(0, _NCH // _NBUF, unroll=2,
                                      carry=(buf0,) * _NBUF)(group)

            def pairmerge(a, b):
                a0, a1, ai0, ai1 = a
                b0, b1, bi0, bi1 = b
                rb0 = lax.rev(b0, (0,))
                rb1 = lax.rev(b1, (0,))
                rbi0 = lax.rev(bi0, (0,))
                rbi1 = lax.rev(bi1, (0,))
                mm0 = a0 < rb1
                m0 = jnp.where(mm0, a0, rb1)
                mi0 = jnp.where(mm0, ai0, rbi1)
                mm1 = a1 < rb0
                m1 = jnp.where(mm1, a1, rb0)
                mi1 = jnp.where(mm1, ai1, rbi0)
                cc = m0 < m1
                lo = jnp.where(cc, m0, m1)
                loi = jnp.where(cc, mi0, mi1)
                hi = jnp.where(cc, m1, m0)
                hii = jnp.where(cc, mi1, mi0)
                s0n, i0n = plsc.sort_key_val(lo, loi)
                s1n, i1n = plsc.sort_key_val(hi, hii)
                return (s0n, s1n, i0n, i1n)

            mA = pairmerge(bufs[0], bufs[1])
            mB = pairmerge(bufs[2], bufs[3])
            s0, s1, i0, i1 = pairmerge(mA, mB)

            base = gp * (_K * 3)
            pos0 = base + 3 * iota
            nx0 = plsc.load_gather(x_v, [i0]) - gx
            ny0 = plsc.load_gather(y_v, [i0]) - gy
            nz0 = plsc.load_gather(z_v, [i0]) - gz
            plsc.store_scatter(nout_v, [pos0], nx0)
            plsc.store_scatter(nout_v, [pos0 + 1], ny0)
            plsc.store_scatter(nout_v, [pos0 + 2], nz0)
            pos1 = base + 3 * _L + 3 * iota
            nx1 = plsc.load_gather(x_v, [i1]) - gx
            ny1 = plsc.load_gather(y_v, [i1]) - gy
            nz1 = plsc.load_gather(z_v, [i1]) - gz
            plsc.store_scatter(nout_v, [pos1], nx1)
            plsc.store_scatter(nout_v, [pos1 + 1], ny1)
            plsc.store_scatter(nout_v, [pos1 + 2], nz1)
            cval = jnp.where(iota == 0, gx, jnp.where(iota == 1, gy, gz))
            plsc.store_scatter(cout_v, [3 * gp + iota], cval,
                               mask=iota < jnp.int32(3))
            return carry0

        lax.fori_loop(0, _G, knn_one, 0)

        pltpu.sync_copy(nout_v, nbr_hbm.at[b])
        pltpu.sync_copy(cout_v, cen_hbm.at[b])
        return carry0

    lax.fori_loop(0, _BPW, batch_body, 0)


import functools


@functools.cache
def _build_sc_call():
    mesh = plsc.VectorSubcoreMesh(core_axis_name="c", subcore_axis_name="s",
                                  num_cores=_NC, num_subcores=_NS)
    return pl.kernel(
        _sc_body,
        compiler_params=pltpu.CompilerParams(needs_layout_passes=False),
        out_type=(jax.ShapeDtypeStruct((_B, _G * _K * 3), jnp.float32),
                  jax.ShapeDtypeStruct((_B, _G * 3), jnp.float32)),
        mesh=mesh,
        scratch_types=[
            pltpu.VMEM((_N,), jnp.float32),
            pltpu.VMEM((_N,), jnp.float32),
            pltpu.VMEM((_N,), jnp.float32),
            pltpu.VMEM((_N,), jnp.float32),
            pltpu.VMEM((_N,), jnp.float32),
            pltpu.VMEM((_G,), jnp.float32),
            pltpu.VMEM((_G,), jnp.float32),
            pltpu.VMEM((_G,), jnp.float32),
            pltpu.VMEM((_G,), jnp.float32),
            pltpu.VMEM((_G,), jnp.float32),
            pltpu.VMEM((_G,), jnp.int32),
            pltpu.VMEM((_N,), jnp.float32),
            pltpu.VMEM((_N,), jnp.float32),
            pltpu.VMEM((_N,), jnp.float32),
            pltpu.VMEM((_G,), jnp.float32),
            pltpu.VMEM((_G,), jnp.float32),
            pltpu.VMEM((_G,), jnp.float32),
            pltpu.VMEM((_G * _K * 3,), jnp.float32),
            pltpu.VMEM((_G * 3,), jnp.float32),
        ],
    )


def kernel(xyz):
    x = xyz[:, :, 0]
    y = xyz[:, :, 1]
    z = xyz[:, :, 2]
    nbr, cen = _build_sc_call()(x, y, z)
    return (nbr.reshape(_B, _G, _K, 3), cen.reshape(_B, _G, 3))
